# baseline (speedup 1.0000x reference)
import functools

import jax
import jax.numpy as jnp
from jax import lax
from jax.experimental import pallas as pl
from jax.experimental.pallas import tpu as pltpu
from jax.experimental.pallas import tpu_sc as plsc

BATCH = 16384
EMBED_DIM = 32
HIDDEN_DIM = 64
EPS = 1e-5

NUM_CORES = 2
NUM_SUBCORES = 16
NUM_WORKERS = NUM_CORES * NUM_SUBCORES
ROWS_PER_WORKER = BATCH // NUM_WORKERS
GATHER_CHUNK = 128
CHUNKS_PER_WORKER = ROWS_PER_WORKER // GATHER_CHUNK


def _sc_gather(table, ids2d):
    mesh = plsc.VectorSubcoreMesh(core_axis_name="c", subcore_axis_name="s")

    @functools.partial(
        pl.kernel,
        mesh=mesh,
        compiler_params=pltpu.CompilerParams(use_tc_tiling_on_sc=False),
        out_type=jax.ShapeDtypeStruct((BATCH, EMBED_DIM), jnp.float32),
        scratch_types=[
            pltpu.VMEM((CHUNKS_PER_WORKER, GATHER_CHUNK), jnp.int32),
            pltpu.VMEM((ROWS_PER_WORKER, EMBED_DIM), jnp.float32),
            pltpu.SemaphoreType.DMA,
        ],
    )
    def gather_kernel(table_hbm, idx_hbm, out_hbm, idx_v, rows_v, sem):
        wid = lax.axis_index("s") * NUM_CORES + lax.axis_index("c")
        pltpu.sync_copy(idx_hbm.at[pl.ds(wid * CHUNKS_PER_WORKER, CHUNKS_PER_WORKER)], idx_v)
        copies = []
        for j in range(CHUNKS_PER_WORKER):
            copies.append(
                pltpu.async_copy(
                    table_hbm.at[idx_v.at[j]],
                    rows_v.at[pl.ds(j * GATHER_CHUNK, GATHER_CHUNK)],
                    sem,
                )
            )
        for c in copies:
            c.wait()
        pltpu.sync_copy(rows_v, out_hbm.at[pl.ds(wid * ROWS_PER_WORKER, ROWS_PER_WORKER)])

    return gather_kernel(table, ids2d)


_MLP_BLOCK = 2048


def _mlp_body(emb_ref, w_ref, b_ref, g_ref, be_ref, out_ref):
    h = lax.dot_general(
        emb_ref[...],
        w_ref[...],
        (((1,), (0,)), ((), ())),
        precision=lax.Precision.HIGHEST,
        preferred_element_type=jnp.float32,
    )
    h = h + b_ref[...]
    mu = jnp.mean(h, axis=1, keepdims=True)
    var = jnp.mean((h - mu) ** 2, axis=1, keepdims=True)
    hn = (h - mu) * lax.rsqrt(var + EPS)
    out_ref[...] = jnp.maximum(hn * g_ref[...] + be_ref[...], 0.0)


def _tc_mlp(emb, W, b, gamma, beta):
    grid = (BATCH // _MLP_BLOCK,)
    return pl.pallas_call(
        _mlp_body,
        grid=grid,
        in_specs=[
            pl.BlockSpec((_MLP_BLOCK, EMBED_DIM), lambda i: (i, 0)),
            pl.BlockSpec((EMBED_DIM, HIDDEN_DIM), lambda i: (0, 0)),
            pl.BlockSpec((1, HIDDEN_DIM), lambda i: (0, 0)),
            pl.BlockSpec((1, HIDDEN_DIM), lambda i: (0, 0)),
            pl.BlockSpec((1, HIDDEN_DIM), lambda i: (0, 0)),
        ],
        out_specs=pl.BlockSpec((_MLP_BLOCK, HIDDEN_DIM), lambda i: (i, 0)),
        out_shape=jax.ShapeDtypeStruct((BATCH, HIDDEN_DIM), jnp.float32),
    )(emb, W, b, gamma, beta)


def kernel(task_ids, table, W, b, gamma, beta):
    ids2d = task_ids.reshape(BATCH // GATHER_CHUNK, GATHER_CHUNK).astype(jnp.int32)
    emb = _sc_gather(table, ids2d)
    return _tc_mlp(
        emb,
        W,
        b.reshape(1, HIDDEN_DIM),
        gamma.reshape(1, HIDDEN_DIM),
        beta.reshape(1, HIDDEN_DIM),
    )


# speedup vs baseline: 1.5818x; 1.5818x over previous
import functools

import jax
import jax.numpy as jnp
from jax import lax
from jax.experimental import pallas as pl
from jax.experimental.pallas import tpu as pltpu
from jax.experimental.pallas import tpu_sc as plsc

BATCH = 16384
EMBED_DIM = 32
HIDDEN_DIM = 64
EPS = 1e-5

NUM_CORES = 2
NUM_SUBCORES = 16
NUM_WORKERS = NUM_CORES * NUM_SUBCORES
ROWS_PER_WORKER = BATCH // NUM_WORKERS
GATHER_CHUNK = 128
CHUNKS_PER_WORKER = ROWS_PER_WORKER // GATHER_CHUNK


_DMA_LAG = 64


def _sc_gather(table, ids2d):
    mesh = plsc.VectorSubcoreMesh(core_axis_name="c", subcore_axis_name="s")

    @functools.partial(
        pl.kernel,
        mesh=mesh,
        out_type=jax.ShapeDtypeStruct((BATCH, EMBED_DIM), jnp.float32),
        scratch_types=[
            pltpu.VMEM((ROWS_PER_WORKER,), jnp.int32),
            pltpu.VMEM((ROWS_PER_WORKER, EMBED_DIM), jnp.float32),
            pltpu.SemaphoreType.DMA,
            pltpu.SemaphoreType.DMA,
        ],
    )
    def gather_kernel(table_hbm, idx_hbm, out_hbm, idx_v, rows_v, sem_i, sem):
        wid = lax.axis_index("s") * NUM_CORES + lax.axis_index("c")
        pltpu.async_copy(idx_hbm.at[wid], idx_v, sem_i).wait()

        def fire_group(base):
            v = idx_v[pl.ds(base, 16)]
            for j in range(16):
                pltpu.async_copy(
                    table_hbm.at[pl.ds(v[j], 1)], rows_v.at[pl.ds(base + j, 1)], sem
                )

        def wait_group():
            for _ in range(16):
                pltpu.make_async_copy(
                    table_hbm.at[pl.ds(0, 1)], rows_v.at[pl.ds(0, 1)], sem
                ).wait()

        fire_group(0)

        @pl.loop(1, ROWS_PER_WORKER // 16)
        def _(g):
            fire_group(g * 16)
            wait_group()

        wait_group()

        pltpu.sync_copy(rows_v, out_hbm.at[pl.ds(wid * ROWS_PER_WORKER, ROWS_PER_WORKER)])

    return gather_kernel(table, ids2d)


_MLP_BLOCK = 2048


def _mlp_body(emb_ref, w_ref, b_ref, g_ref, be_ref, out_ref):
    h = lax.dot_general(
        emb_ref[...],
        w_ref[...],
        (((1,), (0,)), ((), ())),
        precision=lax.Precision.HIGHEST,
        preferred_element_type=jnp.float32,
    )
    h = h + b_ref[...]
    mu = jnp.mean(h, axis=1, keepdims=True)
    var = jnp.mean((h - mu) ** 2, axis=1, keepdims=True)
    hn = (h - mu) * lax.rsqrt(var + EPS)
    out_ref[...] = jnp.maximum(hn * g_ref[...] + be_ref[...], 0.0)


def _tc_mlp(emb, W, b, gamma, beta):
    grid = (BATCH // _MLP_BLOCK,)
    return pl.pallas_call(
        _mlp_body,
        grid=grid,
        in_specs=[
            pl.BlockSpec((_MLP_BLOCK, EMBED_DIM), lambda i: (i, 0)),
            pl.BlockSpec((EMBED_DIM, HIDDEN_DIM), lambda i: (0, 0)),
            pl.BlockSpec((1, HIDDEN_DIM), lambda i: (0, 0)),
            pl.BlockSpec((1, HIDDEN_DIM), lambda i: (0, 0)),
            pl.BlockSpec((1, HIDDEN_DIM), lambda i: (0, 0)),
        ],
        out_specs=pl.BlockSpec((_MLP_BLOCK, HIDDEN_DIM), lambda i: (i, 0)),
        out_shape=jax.ShapeDtypeStruct((BATCH, HIDDEN_DIM), jnp.float32),
    )(emb, W, b, gamma, beta)


def kernel(task_ids, table, W, b, gamma, beta):
    ids2d = task_ids.reshape(NUM_WORKERS, ROWS_PER_WORKER).astype(jnp.int32)
    emb = _sc_gather(table, ids2d)
    return _tc_mlp(
        emb,
        W,
        b.reshape(1, HIDDEN_DIM),
        gamma.reshape(1, HIDDEN_DIM),
        beta.reshape(1, HIDDEN_DIM),
    )
